# Initial kernel scaffold; baseline (speedup 1.0000x reference)
import jax
import jax.numpy as jnp
from jax.experimental import pallas as pl


def kernel(node, relation, node_emb, rela_emb, adj_node, adj_rela, W, b):
    raise NotImplementedError("write your pallas kernel here")


# speedup vs baseline: 11.8010x; 11.8010x over previous
import jax
import jax.numpy as jnp
from jax import lax
from jax.experimental import pallas as pl
from jax.experimental.pallas import tpu as pltpu
from jax.experimental.pallas import tpu_sc as plsc

_NC = 2
_NS = 16
_NW = _NC * _NS
_NB = 16
_F32 = jnp.float32


def _prep_body(re_ref, rel_ref, ccol_ref, ecol_ref, rt_ref):
    re = re_ref[...]
    nrm = re / (jnp.sqrt(jnp.sum(re * re, axis=1, keepdims=True)) + 1e-8)
    cmat = jnp.abs(
        lax.dot_general(nrm, nrm, (((1,), (1,)), ((), ())),
                        preferred_element_type=_F32))
    rel = rel_ref[...]
    oh = (rel == lax.broadcasted_iota(
        jnp.int32, (rel.shape[0], cmat.shape[0]), 1)).astype(_F32)
    ccol = lax.dot_general(oh, cmat, (((1,), (1,)), ((), ())),
                           preferred_element_type=_F32)
    ccol_ref[...] = ccol
    ecol_ref[...] = jnp.exp(ccol)
    rt_ref[...] = lax.dot_general(oh, re, (((1,), (0,)), ((), ())),
                                  preferred_element_type=_F32)


def _sc_body(node_ref, adjn_ref, adjr_ref, emb_ref, ecol_ref,
             g_out, r1_out,
             node_v, n1_v, r1_v, n2_v, r2_v, ecol_v, rows_v, acc_v,
             sem_p, sem_h, sem_a, sem_b):
    D = emb_ref.shape[1]
    bpw = node_v.shape[0]
    nch = D // 16
    wid = lax.axis_index("s") * _NC + lax.axis_index("c")
    base = wid * bpw

    pltpu.sync_copy(node_ref.at[pl.ds(base, bpw)], node_v)
    pltpu.sync_copy(ecol_ref.at[pl.ds(base * _NB, bpw * _NB)], ecol_v)
    for c0 in range(0, bpw, _NB):
        chunk = node_v[pl.ds(c0, _NB)]
        hs = []
        for k in range(_NB):
            nid = chunk[k]
            q = (c0 + k) * _NB
            hs.append(pltpu.async_copy(adjn_ref.at[pl.ds(nid * _NB, _NB)],
                                       n1_v.at[pl.ds(q, _NB)], sem_p))
            hs.append(pltpu.async_copy(adjr_ref.at[pl.ds(nid * _NB, _NB)],
                                       r1_v.at[pl.ds(q, _NB)], sem_p))
        for h in hs:
            h.wait()
    pltpu.sync_copy(r1_v, r1_out.at[pl.ds(base * _NB, bpw * _NB)])

    def h2_issue(g):
        n1chunk = n1_v[pl.ds(g * _NB, _NB)]
        for i in range(_NB):
            nid = n1chunk[i]
            q = (g * _NB + i) * _NB
            pltpu.async_copy(adjn_ref.at[pl.ds(nid * _NB, _NB)],
                             n2_v.at[pl.ds(q, _NB)], sem_h)
            pltpu.async_copy(adjr_ref.at[pl.ds(nid * _NB, _NB)],
                             r2_v.at[pl.ds(q, _NB)], sem_h)

    def h2_wait_group():
        for _ in range(_NB):
            pltpu.make_async_copy(adjn_ref.at[pl.ds(0, _NB)],
                                  n2_v.at[pl.ds(0, _NB)], sem_h).wait()
            pltpu.make_async_copy(adjr_ref.at[pl.ds(0, _NB)],
                                  r2_v.at[pl.ds(0, _NB)], sem_h).wait()

    h2_issue(0)

    def h2_loop(g, carry):
        h2_issue(g)
        h2_wait_group()
        return carry

    lax.fori_loop(1, bpw, h2_loop, 0)
    h2_wait_group()

    sems = (sem_a, sem_b)

    def issue(g, sub):
        for i in range(_NB):
            pltpu.async_copy(
                emb_ref.at[n2_v.at[pl.ds((g * _NB + i) * _NB, _NB)]],
                rows_v.at[sub, i], sems[sub])

    def drain(g, sub):
        for i in range(_NB):
            pltpu.make_async_copy(
                emb_ref.at[n2_v.at[pl.ds((g * _NB + i) * _NB, _NB)]],
                rows_v.at[sub, i], sems[sub]).wait()

    def compute(g, sub):
        erow = ecol_v[pl.ds(g * _NB, _NB)]
        ers = [erow[r] for r in range(_NB)]

        def pair(i, carry):
            r2row = r2_v[pl.ds((g * _NB + i) * _NB, _NB)]
            e = jnp.zeros((16,), _F32)
            for r in range(_NB):
                e = e + jnp.where(r2row == r, ers[r], 0.0)
            svec = jnp.zeros((16,), _F32)
            accs = [jnp.zeros((16,), _F32) for _ in range(nch)]
            for j in range(_NB):
                ej = e[j]
                svec = svec + ej
                for c in range(nch):
                    accs[c] = accs[c] + ej * rows_v[sub, i, j,
                                                   pl.ds(c * 16, 16)]
            winv = 1.0 / svec
            for c in range(nch):
                acc_v[sub, i, pl.ds(c * 16, 16)] = accs[c] * winv
            return carry

        lax.fori_loop(0, _NB, pair, 0)
        pltpu.sync_copy(acc_v.at[sub],
                        g_out.at[pl.ds((base + g) * _NB, _NB)])

    issue(0, 0)
    issue(1, 1)

    def outer(it, carry):
        g = it * 2
        for sub in range(2):
            gc = g + sub
            drain(gc, sub)
            compute(gc, sub)
            issue(gc + 2, sub)
        return carry

    lax.fori_loop(0, bpw // 2 - 1, outer, 0)
    for sub in range(2):
        gc = bpw - 2 + sub
        drain(gc, sub)
        compute(gc, sub)


def _finish_body(g_ref, r1_ref, ccol_ref, rt_ref,
                 w0_ref, w1_ref, b0_ref, b1_ref, out_ref):
    Bb = rt_ref.shape[0]
    D = rt_ref.shape[1]
    R = ccol_ref.shape[1]
    G = g_ref[...]
    G3 = G.reshape(Bb, _NB, D)
    rt = rt_ref[...]
    proj = (G3 * rt[:, None, :]).reshape(Bb * _NB, D)
    v1 = jnp.tanh(
        lax.dot_general(proj, w0_ref[...], (((1,), (0,)), ((), ())),
                        preferred_element_type=_F32) + b0_ref[...])
    v13 = v1.reshape(Bb, _NB, D)
    r1 = r1_ref[...]
    ccol = ccol_ref[...]
    sc1 = jnp.zeros(r1.shape, _F32)
    for r in range(R):
        sc1 = sc1 + jnp.where(r1 == r, ccol[:, r:r + 1], 0.0)
    e1 = jnp.exp(sc1)
    att = e1 / jnp.sum(e1, axis=1, keepdims=True)
    agg = jnp.sum(v13 * att[:, :, None], axis=1)
    out_ref[...] = jnp.tanh(
        lax.dot_general(agg, w1_ref[...], (((1,), (0,)), ((), ())),
                        preferred_element_type=_F32) + b1_ref[...])


def _prep_call(rela_emb, rel2d):
    B = rel2d.shape[0]
    R, D = rela_emb.shape
    return pl.pallas_call(
        _prep_body,
        out_shape=[
            jax.ShapeDtypeStruct((B, R), _F32),
            jax.ShapeDtypeStruct((B, R), _F32),
            jax.ShapeDtypeStruct((B, D), _F32),
        ],
    )(rela_emb, rel2d)


def _sc_call(node, adj_node, adj_rela, node_emb, ecol):
    B = node.shape[0]
    D = node_emb.shape[1]
    bpw = B // _NW
    fn = pl.kernel(
        _sc_body,
        out_type=[
            jax.ShapeDtypeStruct((B * _NB, D), _F32),
            jax.ShapeDtypeStruct((B * _NB,), jnp.int32),
        ],
        scratch_types=[
            pltpu.VMEM((bpw,), jnp.int32),
            pltpu.VMEM((bpw * _NB,), jnp.int32),
            pltpu.VMEM((bpw * _NB,), jnp.int32),
            pltpu.VMEM((bpw * _NB * _NB,), jnp.int32),
            pltpu.VMEM((bpw * _NB * _NB,), jnp.int32),
            pltpu.VMEM((bpw * _NB,), _F32),
            pltpu.VMEM((2, _NB, _NB, D), _F32),
            pltpu.VMEM((2, _NB, D), _F32),
            pltpu.SemaphoreType.DMA,
            pltpu.SemaphoreType.DMA,
            pltpu.SemaphoreType.DMA,
            pltpu.SemaphoreType.DMA,
        ],
        mesh=plsc.VectorSubcoreMesh(core_axis_name="c", subcore_axis_name="s"),
    )
    return fn(node, adj_node, adj_rela, node_emb, ecol)


def _finish_call(gagg, r1, ccol, rt, w0, w1, b0, b1):
    B, D = rt.shape
    R = ccol.shape[1]
    nblk = 4
    bb = B // nblk
    return pl.pallas_call(
        _finish_body,
        grid=(nblk,),
        in_specs=[
            pl.BlockSpec((bb * _NB, D), lambda i: (i, 0)),
            pl.BlockSpec((bb, _NB), lambda i: (i, 0)),
            pl.BlockSpec((bb, R), lambda i: (i, 0)),
            pl.BlockSpec((bb, D), lambda i: (i, 0)),
            pl.BlockSpec((D, D), lambda i: (0, 0)),
            pl.BlockSpec((D, D), lambda i: (0, 0)),
            pl.BlockSpec((1, D), lambda i: (0, 0)),
            pl.BlockSpec((1, D), lambda i: (0, 0)),
        ],
        out_specs=pl.BlockSpec((bb, D), lambda i: (i, 0)),
        out_shape=jax.ShapeDtypeStruct((B, D), _F32),
    )(gagg, r1, ccol, rt, w0, w1, b0, b1)


def kernel(node, relation, node_emb, rela_emb, adj_node, adj_rela, W, b):
    B = node.shape[0]
    D = node_emb.shape[1]
    ccol, ecol, rt = _prep_call(rela_emb, relation.reshape(B, 1))
    gagg, r1 = _sc_call(node, adj_node.reshape(-1), adj_rela.reshape(-1),
                        node_emb, ecol.reshape(-1))
    r1 = r1.reshape(B, _NB)
    return _finish_call(gagg, r1, ccol, rt, W[0], W[1],
                        b[0].reshape(1, D), b[1].reshape(1, D))


# speedup vs baseline: 13.6396x; 1.1558x over previous
import jax
import jax.numpy as jnp
from jax import lax
from jax.experimental import pallas as pl
from jax.experimental.pallas import tpu as pltpu
from jax.experimental.pallas import tpu_sc as plsc

_NC = 2
_NS = 16
_NW = _NC * _NS
_NB = 16
_F32 = jnp.float32


def _prep_body(re_ref, rel_ref, ccol_ref, ecol_ref, rt_ref):
    re = re_ref[...]
    nrm = re / (jnp.sqrt(jnp.sum(re * re, axis=1, keepdims=True)) + 1e-8)
    cmat = jnp.abs(
        lax.dot_general(nrm, nrm, (((1,), (1,)), ((), ())),
                        preferred_element_type=_F32))
    rel = rel_ref[...]
    oh = (rel == lax.broadcasted_iota(
        jnp.int32, (rel.shape[0], cmat.shape[0]), 1)).astype(_F32)
    ccol = lax.dot_general(oh, cmat, (((1,), (1,)), ((), ())),
                           preferred_element_type=_F32)
    ccol_ref[...] = ccol
    ecol_ref[...] = jnp.exp(ccol)
    rt_ref[...] = lax.dot_general(oh, re, (((1,), (0,)), ((), ())),
                                  preferred_element_type=_F32)


def _sc_body(node_ref, adj_ref, emb_ref, ecol_ref,
             g_out, r1_out,
             node_v, nr1_v, r1_v, n2r2_v, ecol_v, rows_v, acc_v,
             sem_p, sem_h, sem_a, sem_b):
    D = emb_ref.shape[1]
    bpw = node_v.shape[0]
    nch = D // 16
    wid = lax.axis_index("s") * _NC + lax.axis_index("c")
    base = wid * bpw

    pltpu.sync_copy(node_ref.at[pl.ds(base, bpw)], node_v)
    pltpu.sync_copy(ecol_ref.at[pl.ds(base * _NB, bpw * _NB)], ecol_v)
    for c0 in range(0, bpw, _NB):
        chunk = node_v[pl.ds(c0, _NB)]
        hs = []
        for k in range(_NB):
            nid = chunk[k]
            q = (c0 + k) * 2 * _NB
            hs.append(pltpu.async_copy(
                adj_ref.at[pl.ds(nid * 2 * _NB, 2 * _NB)],
                nr1_v.at[pl.ds(q, 2 * _NB)], sem_p))
        for h in hs:
            h.wait()
    for bl in range(bpw):
        r1_v[pl.ds(bl * _NB, _NB)] = nr1_v[pl.ds(bl * 2 * _NB + _NB, _NB)]
    pltpu.sync_copy(r1_v, r1_out.at[pl.ds(base * _NB, bpw * _NB)])

    def h2_issue(g):
        n1chunk = nr1_v[pl.ds(g * 2 * _NB, _NB)]
        for i in range(_NB):
            nid = n1chunk[i]
            q = (g * _NB + i) * 2 * _NB
            pltpu.async_copy(adj_ref.at[pl.ds(nid * 2 * _NB, 2 * _NB)],
                             n2r2_v.at[pl.ds(q, 2 * _NB)], sem_h)

    def h2_wait_group():
        for _ in range(_NB):
            pltpu.make_async_copy(adj_ref.at[pl.ds(0, 2 * _NB)],
                                  n2r2_v.at[pl.ds(0, 2 * _NB)], sem_h).wait()

    h2_issue(0)

    def h2_loop(g, carry):
        h2_issue(g)
        h2_wait_group()
        return carry

    lax.fori_loop(1, bpw, h2_loop, 0)
    h2_wait_group()

    sems = (sem_a, sem_b)

    def issue(g, sub):
        for i in range(_NB):
            pltpu.async_copy(
                emb_ref.at[n2r2_v.at[pl.ds((g * _NB + i) * 2 * _NB, _NB)]],
                rows_v.at[sub, i], sems[sub])

    def drain(g, sub):
        for i in range(_NB):
            pltpu.make_async_copy(
                emb_ref.at[n2r2_v.at[pl.ds((g * _NB + i) * 2 * _NB, _NB)]],
                rows_v.at[sub, i], sems[sub]).wait()

    def compute(g, sub):
        erow = ecol_v[pl.ds(g * _NB, _NB)]
        ers = [erow[r] for r in range(_NB)]

        def pair(i, carry):
            r2row = n2r2_v[pl.ds((g * _NB + i) * 2 * _NB + _NB, _NB)]
            e = jnp.zeros((16,), _F32)
            for r in range(_NB):
                e = e + jnp.where(r2row == r, ers[r], 0.0)
            svec = jnp.zeros((16,), _F32)
            accs = [jnp.zeros((16,), _F32) for _ in range(nch)]
            for j in range(_NB):
                ej = e[j]
                svec = svec + ej
                for c in range(nch):
                    accs[c] = accs[c] + ej * rows_v[sub, i, j,
                                                   pl.ds(c * 16, 16)]
            winv = 1.0 / svec
            for c in range(nch):
                acc_v[sub, i, pl.ds(c * 16, 16)] = accs[c] * winv
            return carry

        lax.fori_loop(0, _NB, pair, 0)
        pltpu.sync_copy(acc_v.at[sub],
                        g_out.at[pl.ds((base + g) * _NB, _NB)])

    issue(0, 0)
    issue(1, 1)

    def outer(it, carry):
        g = it * 2
        for sub in range(2):
            gc = g + sub
            drain(gc, sub)
            compute(gc, sub)
            issue(gc + 2, sub)
        return carry

    lax.fori_loop(0, bpw // 2 - 1, outer, 0)
    for sub in range(2):
        gc = bpw - 2 + sub
        drain(gc, sub)
        compute(gc, sub)


def _finish_body(g_ref, r1_ref, ccol_ref, rt_ref,
                 w0_ref, w1_ref, b0_ref, b1_ref, out_ref):
    Bb = rt_ref.shape[0]
    D = rt_ref.shape[1]
    R = ccol_ref.shape[1]
    G = g_ref[...]
    G3 = G.reshape(Bb, _NB, D)
    rt = rt_ref[...]
    proj = (G3 * rt[:, None, :]).reshape(Bb * _NB, D)
    v1 = jnp.tanh(
        lax.dot_general(proj, w0_ref[...], (((1,), (0,)), ((), ())),
                        preferred_element_type=_F32) + b0_ref[...])
    v13 = v1.reshape(Bb, _NB, D)
    r1 = r1_ref[...]
    ccol = ccol_ref[...]
    sc1 = jnp.zeros(r1.shape, _F32)
    for r in range(R):
        sc1 = sc1 + jnp.where(r1 == r, ccol[:, r:r + 1], 0.0)
    e1 = jnp.exp(sc1)
    att = e1 / jnp.sum(e1, axis=1, keepdims=True)
    agg = jnp.sum(v13 * att[:, :, None], axis=1)
    out_ref[...] = jnp.tanh(
        lax.dot_general(agg, w1_ref[...], (((1,), (0,)), ((), ())),
                        preferred_element_type=_F32) + b1_ref[...])


def _prep_call(rela_emb, rel2d):
    B = rel2d.shape[0]
    R, D = rela_emb.shape
    return pl.pallas_call(
        _prep_body,
        out_shape=[
            jax.ShapeDtypeStruct((B, R), _F32),
            jax.ShapeDtypeStruct((B, R), _F32),
            jax.ShapeDtypeStruct((B, D), _F32),
        ],
    )(rela_emb, rel2d)


def _sc_call(node, adj, node_emb, ecol):
    B = node.shape[0]
    D = node_emb.shape[1]
    bpw = B // _NW
    fn = pl.kernel(
        _sc_body,
        out_type=[
            jax.ShapeDtypeStruct((B * _NB, D), _F32),
            jax.ShapeDtypeStruct((B * _NB,), jnp.int32),
        ],
        scratch_types=[
            pltpu.VMEM((bpw,), jnp.int32),
            pltpu.VMEM((bpw * 2 * _NB,), jnp.int32),
            pltpu.VMEM((bpw * _NB,), jnp.int32),
            pltpu.VMEM((bpw * _NB * 2 * _NB,), jnp.int32),
            pltpu.VMEM((bpw * _NB,), _F32),
            pltpu.VMEM((2, _NB, _NB, D), _F32),
            pltpu.VMEM((2, _NB, D), _F32),
            pltpu.SemaphoreType.DMA,
            pltpu.SemaphoreType.DMA,
            pltpu.SemaphoreType.DMA,
            pltpu.SemaphoreType.DMA,
        ],
        mesh=plsc.VectorSubcoreMesh(core_axis_name="c", subcore_axis_name="s"),
    )
    return fn(node, adj, node_emb, ecol)


def _finish_call(gagg, r1, ccol, rt, w0, w1, b0, b1):
    B, D = rt.shape
    R = ccol.shape[1]
    nblk = 4
    bb = B // nblk
    return pl.pallas_call(
        _finish_body,
        grid=(nblk,),
        in_specs=[
            pl.BlockSpec((bb * _NB, D), lambda i: (i, 0)),
            pl.BlockSpec((bb, _NB), lambda i: (i, 0)),
            pl.BlockSpec((bb, R), lambda i: (i, 0)),
            pl.BlockSpec((bb, D), lambda i: (i, 0)),
            pl.BlockSpec((D, D), lambda i: (0, 0)),
            pl.BlockSpec((D, D), lambda i: (0, 0)),
            pl.BlockSpec((1, D), lambda i: (0, 0)),
            pl.BlockSpec((1, D), lambda i: (0, 0)),
        ],
        out_specs=pl.BlockSpec((bb, D), lambda i: (i, 0)),
        out_shape=jax.ShapeDtypeStruct((B, D), _F32),
    )(gagg, r1, ccol, rt, w0, w1, b0, b1)


def kernel(node, relation, node_emb, rela_emb, adj_node, adj_rela, W, b):
    B = node.shape[0]
    D = node_emb.shape[1]
    ccol, ecol, rt = _prep_call(rela_emb, relation.reshape(B, 1))
    adj = jnp.concatenate([adj_node, adj_rela], axis=1).reshape(-1)
    gagg, r1 = _sc_call(node, adj, node_emb, ecol.reshape(-1))
    r1 = r1.reshape(B, _NB)
    return _finish_call(gagg, r1, ccol, rt, W[0], W[1],
                        b[0].reshape(1, D), b[1].reshape(1, D))
